# baseline (speedup 1.0000x reference)
import functools

import jax
import jax.numpy as jnp
from jax.experimental import pallas as pl
from jax.experimental.pallas import tpu as pltpu

_HID = 5
_STEPS = 10


def _gnn_kernel(Je_ref, Jo_ref, b_ref, bp_ref, w_bin_ref, wJ_ref, Wh2_ref,
                wbout2_ref, bm1p_ref, W2b_ref, bm2p_ref, W3p_ref, bm3p_ref,
                Wzp_ref, bzp_ref, Wrp_ref, brp_ref, Whp_ref, bhp_ref,
                Wo1h2_ref, Wo1b2_ref, bo1p_ref, Wo2b_ref, bo2p_ref,
                Wo3b_ref, bo3p_ref,
                out_ref, h0_ref, h1_ref, m0_ref, base_ref, *, n_i_tile):
    f32 = jnp.float32
    bf16 = jnp.bfloat16
    Je = Je_ref[0]
    Jo = Jo_ref[0]
    bv = b_ref[0]
    bp = bp_ref[0]
    n = Je.shape[0]
    nh = n // 2
    TI = n_i_tile

    w_bin = w_bin_ref[:]
    wJ = wJ_ref[:]
    Wh2 = Wh2_ref[:]
    wbout2 = wbout2_ref[:]
    bm1p = bm1p_ref[:]
    W2b = W2b_ref[:]
    bm2p = bm2p_ref[:]
    W3p = W3p_ref[:]
    bm3p = bm3p_ref[:]
    Wzp, bzp = Wzp_ref[:], bzp_ref[:]
    Wrp, brp = Wrp_ref[:], brp_ref[:]
    Whp, bhp = Whp_ref[:], bhp_ref[:]

    v = bv @ w_bin
    vv = jnp.concatenate([v, v], axis=1)
    c2 = bp @ wbout2 + bm1p

    for t in range(n // TI):
        i0 = t * TI
        te = Je[i0:i0 + TI][:, :, None] * wJ[0][None, None, :]
        to = Jo[i0:i0 + TI][:, :, None] * wJ[0][None, None, :]
        base_ref[i0:i0 + TI] = (
            jnp.concatenate([te, to], axis=2)
            + vv[i0:i0 + TI, None, :]).astype(bf16)

    def msg_pair(h0p, h1p):
        u2 = jnp.stack([h0p @ Wh2 + c2, h1p @ Wh2 + c2]).astype(bf16)
        s = jnp.zeros((2, nh, 2 * _HID), f32)
        for t in range(n // TI):
            i0 = t * TI
            base = base_ref[i0:i0 + TI]
            x1 = jnp.maximum(u2[:, None, :, :] + base[None], 0)
            x2 = jnp.maximum(
                jnp.dot(x1.reshape(2 * TI * nh, 128), W2b,
                        preferred_element_type=f32) + bm2p, 0.0)
            y = jnp.dot(x2, W3p, preferred_element_type=f32)
            s = s + y.reshape(2, TI, nh, 2 * _HID).sum(axis=1)
        d2 = s.reshape(2 * nh, 2 * _HID) + jnp.float32(n) * bm3p
        return d2[0:nh], d2[nh:2 * nh]

    def gru(hp, m0p, m1p):
        ap = jnp.concatenate([hp, m0p, m1p], axis=1)
        z = jax.nn.sigmoid(ap @ Wzp + bzp)
        r = jax.nn.sigmoid(ap @ Wrp + brp)
        jp = jnp.concatenate([r * hp, m0p, m1p], axis=1)
        hh = jnp.tanh(jp @ Whp + bhp)
        return (1.0 - z) * hp + z * hh

    h0_ref[:] = jnp.zeros((nh, 2 * _HID), f32)
    h1_ref[:] = jnp.zeros((nh, 2 * _HID), f32)
    m0_ref[:] = jnp.zeros((nh, 2 * _HID), f32)

    def step(_, tok):
        h0p, h1p = h0_ref[:], h1_ref[:]
        d0, d1 = msg_pair(h0p, h1p)
        m0p = m0_ref[:] + d0
        m1p = m0p + d1
        m0_ref[:] = m0p
        h0_ref[:] = h1p
        h1_ref[:] = gru(h1p, m0p, m1p)
        return tok

    jax.lax.fori_loop(0, _STEPS, step, 0)
    h1p = h1_ref[:]

    Wo1h2 = Wo1h2_ref[:]
    Wo1b2 = Wo1b2_ref[:]
    bo1p = bo1p_ref[:]
    Wo2b = Wo2b_ref[:]
    bo2p = bo2p_ref[:]
    Wo3b = Wo3b_ref[:]
    bo3p = bo3p_ref[:]
    x = jnp.maximum(h1p @ Wo1h2 + bp @ Wo1b2 + bo1p, 0.0)
    x = jnp.maximum(x @ Wo2b + bo2p, 0.0)
    out_ref[0, :, :] = jax.nn.sigmoid(x @ Wo3b + bo3p)


def _blkdiag(a):
    za = jnp.zeros_like(a)
    return jnp.concatenate(
        [jnp.concatenate([a, za], axis=1),
         jnp.concatenate([za, a], axis=1)], axis=0)


def kernel(J, b, Wm1, bm1, Wm2, bm2, Wm3, bm3, Wz, bz, Wr, br, Wh, bh,
           Wo1, bo1, Wo2, bo2, Wo3, bo3):
    B, n = J.shape[0], J.shape[1]
    nh = n // 2
    f32 = jnp.float32

    Je = J[:, :, 0::2]
    Jo = J[:, :, 1::2]
    bp = b.reshape(B, nh, 2)

    def pair2(row):
        return jnp.concatenate([row, row], axis=1)

    w_bin = Wm1[_HID:_HID + 1]
    wJ = Wm1[_HID + 2:_HID + 3] - Wm1[_HID + 3:_HID + 4]
    Wh2 = _blkdiag(Wm1[0:_HID])
    wbout2 = _blkdiag(Wm1[_HID + 1:_HID + 2])
    bm1p = pair2(bm1.reshape(1, -1))
    W2b = _blkdiag(Wm2).astype(jnp.bfloat16)
    bm2p = pair2(bm2.reshape(1, -1))
    W3p = _blkdiag(Wm3)
    bm3p = pair2(bm3.reshape(1, -1))

    def gru_pack(W):
        return jnp.concatenate([_blkdiag(W[0:_HID]),
                                _blkdiag(W[_HID:2 * _HID]),
                                _blkdiag(W[2 * _HID:])], axis=0)

    Wzp, bzp = gru_pack(Wz), pair2(bz.reshape(1, -1))
    Wrp, brp = gru_pack(Wr), pair2(br.reshape(1, -1))
    Whp, bhp = gru_pack(Wh), pair2(bh.reshape(1, -1))

    Wo1h2 = _blkdiag(Wo1[0:_HID])
    Wo1b2 = _blkdiag(Wo1[_HID:])
    bo1p = pair2(bo1.reshape(1, -1))
    Wo2b = _blkdiag(Wo2)
    bo2p = pair2(bo2.reshape(1, -1))
    Wo3b = _blkdiag(Wo3)
    bo3p = pair2(bo3.reshape(1, -1))

    weights = (w_bin, wJ, Wh2, wbout2, bm1p, W2b, bm2p, W3p, bm3p,
               Wzp, bzp, Wrp, brp, Whp, bhp,
               Wo1h2, Wo1b2, bo1p, Wo2b, bo2p, Wo3b, bo3p)

    def wspec(w):
        return pl.BlockSpec(w.shape, lambda i: (0,) * w.ndim)

    out = pl.pallas_call(
        functools.partial(_gnn_kernel, n_i_tile=64),
        grid=(B,),
        in_specs=[pl.BlockSpec((1, n, nh), lambda i: (i, 0, 0)),
                  pl.BlockSpec((1, n, nh), lambda i: (i, 0, 0)),
                  pl.BlockSpec((1, n, 1), lambda i: (i, 0, 0)),
                  pl.BlockSpec((1, nh, 2), lambda i: (i, 0, 0))]
                 + [wspec(w) for w in weights],
        out_specs=pl.BlockSpec((1, nh, 2), lambda i: (i, 0, 0)),
        out_shape=jax.ShapeDtypeStruct((B, nh, 2), f32),
        scratch_shapes=[pltpu.VMEM((nh, 2 * _HID), f32),
                        pltpu.VMEM((nh, 2 * _HID), f32),
                        pltpu.VMEM((nh, 2 * _HID), f32),
                        pltpu.VMEM((n, nh, 128), jnp.bfloat16)],
        compiler_params=pltpu.CompilerParams(
            dimension_semantics=("parallel",)),
    )(Je, Jo, b, bp, *weights)
    return out.reshape(B, 1, n)


# speedup vs baseline: 1.0000x; 1.0000x over previous
import functools

import jax
import jax.numpy as jnp
from jax.experimental import pallas as pl
from jax.experimental.pallas import tpu as pltpu

_HID = 5
_STEPS = 10


def _gnn_kernel(Je_ref, Jo_ref, b_ref, bp_ref, w_bin_ref, wJ_ref, Wh2_ref,
                wbout2_ref, bm1p_ref, W2b_ref, bm2p_ref, W3p_ref, bm3p_ref,
                Wzp_ref, bzp_ref, Wrp_ref, brp_ref, Whp_ref, bhp_ref,
                Wo1h2_ref, Wo1b2_ref, bo1p_ref, Wo2b_ref, bo2p_ref,
                Wo3b_ref, bo3p_ref,
                out_ref, h0_ref, h1_ref, m0_ref, base_ref, *, n_i_tile):
    f32 = jnp.float32
    bf16 = jnp.bfloat16
    Je = Je_ref[0]
    Jo = Jo_ref[0]
    bv = b_ref[0]
    bp = bp_ref[0]
    n = Je.shape[0]
    nh = n // 2
    TI = n_i_tile

    w_bin = w_bin_ref[:]
    wJ = wJ_ref[:]
    Wh2 = Wh2_ref[:]
    wbout2 = wbout2_ref[:]
    bm1p = bm1p_ref[:]
    W2b = W2b_ref[:]
    bm2p = bm2p_ref[:]
    W3p = W3p_ref[:]
    bm3p = bm3p_ref[:]
    Wzp, bzp = Wzp_ref[:], bzp_ref[:]
    Wrp, brp = Wrp_ref[:], brp_ref[:]
    Whp, bhp = Whp_ref[:], bhp_ref[:]

    bm2b = bm2p.astype(bf16)
    W3b = W3p.astype(bf16)

    v = bv @ w_bin
    vv = jnp.concatenate([v, v], axis=1)
    c2 = bp @ wbout2 + bm1p

    for t in range(n // TI):
        i0 = t * TI
        te = Je[i0:i0 + TI][:, :, None] * wJ[0][None, None, :]
        to = Jo[i0:i0 + TI][:, :, None] * wJ[0][None, None, :]
        base_ref[i0:i0 + TI] = (
            jnp.concatenate([te, to], axis=2)
            + vv[i0:i0 + TI, None, :]).astype(bf16)

    def msg_pair(h0p, h1p):
        u2 = jnp.stack([h0p @ Wh2 + c2, h1p @ Wh2 + c2]).astype(bf16)
        s = jnp.zeros((2, nh, 2 * _HID), f32)
        for t in range(n // TI):
            i0 = t * TI
            base = base_ref[i0:i0 + TI]
            x1 = jnp.maximum(u2[:, None, :, :] + base[None], 0)
            x2 = jnp.maximum(
                jnp.dot(x1.reshape(2 * TI * nh, 128), W2b,
                        preferred_element_type=f32).astype(bf16) + bm2b, 0)
            y = jnp.dot(x2, W3b, preferred_element_type=f32)
            s = s + y.reshape(2, TI, nh, 2 * _HID).sum(axis=1)
        d2 = s.reshape(2 * nh, 2 * _HID) + jnp.float32(n) * bm3p
        return d2[0:nh], d2[nh:2 * nh]

    def gru(hp, m0p, m1p):
        ap = jnp.concatenate([hp, m0p, m1p], axis=1)
        z = jax.nn.sigmoid(ap @ Wzp + bzp)
        r = jax.nn.sigmoid(ap @ Wrp + brp)
        jp = jnp.concatenate([r * hp, m0p, m1p], axis=1)
        hh = jnp.tanh(jp @ Whp + bhp)
        return (1.0 - z) * hp + z * hh

    h0_ref[:] = jnp.zeros((nh, 2 * _HID), f32)
    h1_ref[:] = jnp.zeros((nh, 2 * _HID), f32)
    m0_ref[:] = jnp.zeros((nh, 2 * _HID), f32)

    def step(_, tok):
        h0p, h1p = h0_ref[:], h1_ref[:]
        d0, d1 = msg_pair(h0p, h1p)
        m0p = m0_ref[:] + d0
        m1p = m0p + d1
        m0_ref[:] = m0p
        h0_ref[:] = h1p
        h1_ref[:] = gru(h1p, m0p, m1p)
        return tok

    jax.lax.fori_loop(0, _STEPS, step, 0)
    h1p = h1_ref[:]

    Wo1h2 = Wo1h2_ref[:]
    Wo1b2 = Wo1b2_ref[:]
    bo1p = bo1p_ref[:]
    Wo2b = Wo2b_ref[:]
    bo2p = bo2p_ref[:]
    Wo3b = Wo3b_ref[:]
    bo3p = bo3p_ref[:]
    x = jnp.maximum(h1p @ Wo1h2 + bp @ Wo1b2 + bo1p, 0.0)
    x = jnp.maximum(x @ Wo2b + bo2p, 0.0)
    out_ref[0, :, :] = jax.nn.sigmoid(x @ Wo3b + bo3p)


def _blkdiag(a):
    za = jnp.zeros_like(a)
    return jnp.concatenate(
        [jnp.concatenate([a, za], axis=1),
         jnp.concatenate([za, a], axis=1)], axis=0)


def kernel(J, b, Wm1, bm1, Wm2, bm2, Wm3, bm3, Wz, bz, Wr, br, Wh, bh,
           Wo1, bo1, Wo2, bo2, Wo3, bo3):
    B, n = J.shape[0], J.shape[1]
    nh = n // 2
    f32 = jnp.float32

    Je = J[:, :, 0::2]
    Jo = J[:, :, 1::2]
    bp = b.reshape(B, nh, 2)

    def pair2(row):
        return jnp.concatenate([row, row], axis=1)

    w_bin = Wm1[_HID:_HID + 1]
    wJ = Wm1[_HID + 2:_HID + 3] - Wm1[_HID + 3:_HID + 4]
    Wh2 = _blkdiag(Wm1[0:_HID])
    wbout2 = _blkdiag(Wm1[_HID + 1:_HID + 2])
    bm1p = pair2(bm1.reshape(1, -1))
    W2b = _blkdiag(Wm2).astype(jnp.bfloat16)
    bm2p = pair2(bm2.reshape(1, -1))
    W3p = _blkdiag(Wm3)
    bm3p = pair2(bm3.reshape(1, -1))

    def gru_pack(W):
        return jnp.concatenate([_blkdiag(W[0:_HID]),
                                _blkdiag(W[_HID:2 * _HID]),
                                _blkdiag(W[2 * _HID:])], axis=0)

    Wzp, bzp = gru_pack(Wz), pair2(bz.reshape(1, -1))
    Wrp, brp = gru_pack(Wr), pair2(br.reshape(1, -1))
    Whp, bhp = gru_pack(Wh), pair2(bh.reshape(1, -1))

    Wo1h2 = _blkdiag(Wo1[0:_HID])
    Wo1b2 = _blkdiag(Wo1[_HID:])
    bo1p = pair2(bo1.reshape(1, -1))
    Wo2b = _blkdiag(Wo2)
    bo2p = pair2(bo2.reshape(1, -1))
    Wo3b = _blkdiag(Wo3)
    bo3p = pair2(bo3.reshape(1, -1))

    weights = (w_bin, wJ, Wh2, wbout2, bm1p, W2b, bm2p, W3p, bm3p,
               Wzp, bzp, Wrp, brp, Whp, bhp,
               Wo1h2, Wo1b2, bo1p, Wo2b, bo2p, Wo3b, bo3p)

    def wspec(w):
        return pl.BlockSpec(w.shape, lambda i: (0,) * w.ndim)

    out = pl.pallas_call(
        functools.partial(_gnn_kernel, n_i_tile=64),
        grid=(B,),
        in_specs=[pl.BlockSpec((1, n, nh), lambda i: (i, 0, 0)),
                  pl.BlockSpec((1, n, nh), lambda i: (i, 0, 0)),
                  pl.BlockSpec((1, n, 1), lambda i: (i, 0, 0)),
                  pl.BlockSpec((1, nh, 2), lambda i: (i, 0, 0))]
                 + [wspec(w) for w in weights],
        out_specs=pl.BlockSpec((1, nh, 2), lambda i: (i, 0, 0)),
        out_shape=jax.ShapeDtypeStruct((B, nh, 2), f32),
        scratch_shapes=[pltpu.VMEM((nh, 2 * _HID), f32),
                        pltpu.VMEM((nh, 2 * _HID), f32),
                        pltpu.VMEM((nh, 2 * _HID), f32),
                        pltpu.VMEM((n, nh, 128), jnp.bfloat16)],
        compiler_params=pltpu.CompilerParams(
            dimension_semantics=("parallel",)),
    )(Je, Jo, b, bp, *weights)
    return out.reshape(B, 1, n)


# speedup vs baseline: 1.2647x; 1.2647x over previous
import functools

import jax
import jax.numpy as jnp
from jax.experimental import pallas as pl
from jax.experimental.pallas import tpu as pltpu

_HID = 5
_STEPS = 10


def _gnn_kernel(Je_ref, Jo_ref, b_ref, bp_ref, w_bin_ref, wJ_ref, Wh2_ref,
                wbout2_ref, bm1p_ref, W2b_ref, bm2p_ref, W3p_ref, bm3p_ref,
                Wzp_ref, bzp_ref, Wrp_ref, brp_ref, Whp_ref, bhp_ref,
                Wo1h2_ref, Wo1b2_ref, bo1p_ref, Wo2b_ref, bo2p_ref,
                Wo3b_ref, bo3p_ref,
                out_ref, h0_ref, h1_ref, m0_ref, base_ref, *, n_i_tile):
    f32 = jnp.float32
    bf16 = jnp.bfloat16
    Je = Je_ref[0]
    Jo = Jo_ref[0]
    bv = b_ref[0]
    bp = bp_ref[0]
    n = Je.shape[0]
    nh = n // 2
    TI = n_i_tile

    w_bin = w_bin_ref[:]
    wJ = wJ_ref[:]
    Wh2 = Wh2_ref[:]
    wbout2 = wbout2_ref[:]
    bm1p = bm1p_ref[:]
    W2b = W2b_ref[:]
    bm2p = bm2p_ref[:]
    W3p = W3p_ref[:]
    bm3p = bm3p_ref[:]
    Wzp, bzp = Wzp_ref[:], bzp_ref[:]
    Wrp, brp = Wrp_ref[:], brp_ref[:]
    Whp, bhp = Whp_ref[:], bhp_ref[:]

    v = bv @ w_bin
    vv = jnp.concatenate([v, v], axis=1)
    c2 = bp @ wbout2 + bm1p

    for t in range(n // TI):
        i0 = t * TI
        te = Je[i0:i0 + TI][:, :, None] * wJ[0][None, None, :]
        to = Jo[i0:i0 + TI][:, :, None] * wJ[0][None, None, :]
        base_ref[i0:i0 + TI] = (
            jnp.concatenate([te, to], axis=2)
            + vv[i0:i0 + TI, None, :])

    def msg_pair(h0p, h1p):
        u2 = jnp.stack([h0p @ Wh2 + c2, h1p @ Wh2 + c2])
        s = jnp.zeros((2, nh, 128), f32)
        for t in range(n // TI):
            i0 = t * TI
            base = base_ref[i0:i0 + TI]
            x1 = jnp.maximum(u2[:, None, :, :] + base[None], 0)
            x2 = jnp.maximum(
                jnp.dot(x1.reshape(2 * TI * nh, 128), W2b,
                        preferred_element_type=f32) + bm2p, 0.0)
            s = s + x2.reshape(2, TI, nh, 128).sum(axis=1)
        d2 = s.reshape(2 * nh, 128) @ W3p + jnp.float32(n) * bm3p
        return d2[0:nh], d2[nh:2 * nh]

    def gru(hp, m0p, m1p):
        ap = jnp.concatenate([hp, m0p, m1p], axis=1)
        z = jax.nn.sigmoid(ap @ Wzp + bzp)
        r = jax.nn.sigmoid(ap @ Wrp + brp)
        jp = jnp.concatenate([r * hp, m0p, m1p], axis=1)
        hh = jnp.tanh(jp @ Whp + bhp)
        return (1.0 - z) * hp + z * hh

    h0_ref[:] = jnp.zeros((nh, 2 * _HID), f32)
    h1_ref[:] = jnp.zeros((nh, 2 * _HID), f32)
    m0_ref[:] = jnp.zeros((nh, 2 * _HID), f32)

    def step(_, tok):
        h0p, h1p = h0_ref[:], h1_ref[:]
        d0, d1 = msg_pair(h0p, h1p)
        m0p = m0_ref[:] + d0
        m1p = m0p + d1
        m0_ref[:] = m0p
        h0_ref[:] = h1p
        h1_ref[:] = gru(h1p, m0p, m1p)
        return tok

    jax.lax.fori_loop(0, _STEPS, step, 0)
    h1p = h1_ref[:]

    Wo1h2 = Wo1h2_ref[:]
    Wo1b2 = Wo1b2_ref[:]
    bo1p = bo1p_ref[:]
    Wo2b = Wo2b_ref[:]
    bo2p = bo2p_ref[:]
    Wo3b = Wo3b_ref[:]
    bo3p = bo3p_ref[:]
    x = jnp.maximum(h1p @ Wo1h2 + bp @ Wo1b2 + bo1p, 0.0)
    x = jnp.maximum(x @ Wo2b + bo2p, 0.0)
    out_ref[0, :, :] = jax.nn.sigmoid(x @ Wo3b + bo3p)


def _blkdiag(a):
    za = jnp.zeros_like(a)
    return jnp.concatenate(
        [jnp.concatenate([a, za], axis=1),
         jnp.concatenate([za, a], axis=1)], axis=0)


def kernel(J, b, Wm1, bm1, Wm2, bm2, Wm3, bm3, Wz, bz, Wr, br, Wh, bh,
           Wo1, bo1, Wo2, bo2, Wo3, bo3):
    B, n = J.shape[0], J.shape[1]
    nh = n // 2
    f32 = jnp.float32

    Je = J[:, :, 0::2]
    Jo = J[:, :, 1::2]
    bp = b.reshape(B, nh, 2)

    def pair2(row):
        return jnp.concatenate([row, row], axis=1)

    w_bin = Wm1[_HID:_HID + 1]
    wJ = Wm1[_HID + 2:_HID + 3] - Wm1[_HID + 3:_HID + 4]
    Wh2 = _blkdiag(Wm1[0:_HID])
    wbout2 = _blkdiag(Wm1[_HID + 1:_HID + 2])
    bm1p = pair2(bm1.reshape(1, -1))
    W2b = _blkdiag(Wm2)
    bm2p = pair2(bm2.reshape(1, -1))
    W3p = _blkdiag(Wm3)
    bm3p = pair2(bm3.reshape(1, -1))

    def gru_pack(W):
        return jnp.concatenate([_blkdiag(W[0:_HID]),
                                _blkdiag(W[_HID:2 * _HID]),
                                _blkdiag(W[2 * _HID:])], axis=0)

    Wzp, bzp = gru_pack(Wz), pair2(bz.reshape(1, -1))
    Wrp, brp = gru_pack(Wr), pair2(br.reshape(1, -1))
    Whp, bhp = gru_pack(Wh), pair2(bh.reshape(1, -1))

    Wo1h2 = _blkdiag(Wo1[0:_HID])
    Wo1b2 = _blkdiag(Wo1[_HID:])
    bo1p = pair2(bo1.reshape(1, -1))
    Wo2b = _blkdiag(Wo2)
    bo2p = pair2(bo2.reshape(1, -1))
    Wo3b = _blkdiag(Wo3)
    bo3p = pair2(bo3.reshape(1, -1))

    weights = (w_bin, wJ, Wh2, wbout2, bm1p, W2b, bm2p, W3p, bm3p,
               Wzp, bzp, Wrp, brp, Whp, bhp,
               Wo1h2, Wo1b2, bo1p, Wo2b, bo2p, Wo3b, bo3p)

    def wspec(w):
        return pl.BlockSpec(w.shape, lambda i: (0,) * w.ndim)

    out = pl.pallas_call(
        functools.partial(_gnn_kernel, n_i_tile=64),
        grid=(B,),
        in_specs=[pl.BlockSpec((1, n, nh), lambda i: (i, 0, 0)),
                  pl.BlockSpec((1, n, nh), lambda i: (i, 0, 0)),
                  pl.BlockSpec((1, n, 1), lambda i: (i, 0, 0)),
                  pl.BlockSpec((1, nh, 2), lambda i: (i, 0, 0))]
                 + [wspec(w) for w in weights],
        out_specs=pl.BlockSpec((1, nh, 2), lambda i: (i, 0, 0)),
        out_shape=jax.ShapeDtypeStruct((B, nh, 2), f32),
        scratch_shapes=[pltpu.VMEM((nh, 2 * _HID), f32),
                        pltpu.VMEM((nh, 2 * _HID), f32),
                        pltpu.VMEM((nh, 2 * _HID), f32),
                        pltpu.VMEM((n, nh, 128), jnp.float32)],
        compiler_params=pltpu.CompilerParams(
            dimension_semantics=("parallel",)),
    )(Je, Jo, b, bp, *weights)
    return out.reshape(B, 1, n)


# speedup vs baseline: 1.3509x; 1.0681x over previous
import functools

import jax
import jax.numpy as jnp
from jax.experimental import pallas as pl
from jax.experimental.pallas import tpu as pltpu

_HID = 5
_STEPS = 10


def _gnn_kernel(Je_ref, Jo_ref, b_ref, bp_ref, w_bin_ref, wJ_ref, Wh2_ref,
                wbout2_ref, bm1p_ref, W2b_ref, bm2p_ref, W3p_ref, bm3p_ref,
                Wzp_ref, bzp_ref, Wrp_ref, brp_ref, Whp_ref, bhp_ref,
                Wo1h2_ref, Wo1b2_ref, bo1p_ref, Wo2b_ref, bo2p_ref,
                Wo3b_ref, bo3p_ref,
                out_ref, h0_ref, h1_ref, m0_ref, base_ref, *, n_i_tile):
    f32 = jnp.float32
    bf16 = jnp.bfloat16
    Je = Je_ref[0]
    Jo = Jo_ref[0]
    bv = b_ref[0]
    bp = bp_ref[0]
    n = Je.shape[0]
    nh = n // 2
    TI = n_i_tile

    w_bin = w_bin_ref[:]
    wJ = wJ_ref[:]
    Wh2 = Wh2_ref[:]
    wbout2 = wbout2_ref[:]
    bm1p = bm1p_ref[:]
    W2b = W2b_ref[:]
    bm2p = bm2p_ref[:]
    W3p = W3p_ref[:]
    bm3p = bm3p_ref[:]
    Wzp, bzp = Wzp_ref[:], bzp_ref[:]
    Wrp, brp = Wrp_ref[:], brp_ref[:]
    Whp, bhp = Whp_ref[:], bhp_ref[:]

    bm2b = bm2p.astype(bf16)

    v = bv @ w_bin
    vv = jnp.concatenate([v, v], axis=1)
    c2 = bp @ wbout2 + bm1p

    for t in range(n // TI):
        i0 = t * TI
        te = Je[i0:i0 + TI][:, :, None] * wJ[0][None, None, :]
        to = Jo[i0:i0 + TI][:, :, None] * wJ[0][None, None, :]
        base_ref[i0:i0 + TI] = (
            jnp.concatenate([te, to], axis=2)
            + vv[i0:i0 + TI, None, :]).astype(bf16)

    def msg_pair(h0p, h1p):
        u2 = jnp.stack([h0p @ Wh2 + c2, h1p @ Wh2 + c2]).astype(bf16)
        s = jnp.zeros((2, nh, 128), f32)
        for t in range(n // TI):
            i0 = t * TI
            base = base_ref[i0:i0 + TI]
            x1 = jnp.maximum(u2[:, None, :, :] + base[None], 0)
            x2 = jnp.maximum(
                jnp.dot(x1.reshape(2 * TI * nh, 128), W2b,
                        preferred_element_type=f32).astype(bf16) + bm2b, 0)
            s = s + x2.reshape(2, TI, nh, 128).astype(f32).sum(axis=1)
        d2 = s.reshape(2 * nh, 128) @ W3p + jnp.float32(n) * bm3p
        return d2[0:nh], d2[nh:2 * nh]

    def gru(hp, m0p, m1p):
        ap = jnp.concatenate([hp, m0p, m1p], axis=1)
        z = jax.nn.sigmoid(ap @ Wzp + bzp)
        r = jax.nn.sigmoid(ap @ Wrp + brp)
        jp = jnp.concatenate([r * hp, m0p, m1p], axis=1)
        hh = jnp.tanh(jp @ Whp + bhp)
        return (1.0 - z) * hp + z * hh

    h0_ref[:] = jnp.zeros((nh, 2 * _HID), f32)
    h1_ref[:] = jnp.zeros((nh, 2 * _HID), f32)
    m0_ref[:] = jnp.zeros((nh, 2 * _HID), f32)

    def step(_, tok):
        h0p, h1p = h0_ref[:], h1_ref[:]
        d0, d1 = msg_pair(h0p, h1p)
        m0p = m0_ref[:] + d0
        m1p = m0p + d1
        m0_ref[:] = m0p
        h0_ref[:] = h1p
        h1_ref[:] = gru(h1p, m0p, m1p)
        return tok

    jax.lax.fori_loop(0, _STEPS, step, 0)
    h1p = h1_ref[:]

    Wo1h2 = Wo1h2_ref[:]
    Wo1b2 = Wo1b2_ref[:]
    bo1p = bo1p_ref[:]
    Wo2b = Wo2b_ref[:]
    bo2p = bo2p_ref[:]
    Wo3b = Wo3b_ref[:]
    bo3p = bo3p_ref[:]
    x = jnp.maximum(h1p @ Wo1h2 + bp @ Wo1b2 + bo1p, 0.0)
    x = jnp.maximum(x @ Wo2b + bo2p, 0.0)
    out_ref[0, :, :] = jax.nn.sigmoid(x @ Wo3b + bo3p)


def _blkdiag(a):
    za = jnp.zeros_like(a)
    return jnp.concatenate(
        [jnp.concatenate([a, za], axis=1),
         jnp.concatenate([za, a], axis=1)], axis=0)


def kernel(J, b, Wm1, bm1, Wm2, bm2, Wm3, bm3, Wz, bz, Wr, br, Wh, bh,
           Wo1, bo1, Wo2, bo2, Wo3, bo3):
    B, n = J.shape[0], J.shape[1]
    nh = n // 2
    f32 = jnp.float32

    Je = J[:, :, 0::2]
    Jo = J[:, :, 1::2]
    bp = b.reshape(B, nh, 2)

    def pair2(row):
        return jnp.concatenate([row, row], axis=1)

    w_bin = Wm1[_HID:_HID + 1]
    wJ = Wm1[_HID + 2:_HID + 3] - Wm1[_HID + 3:_HID + 4]
    Wh2 = _blkdiag(Wm1[0:_HID])
    wbout2 = _blkdiag(Wm1[_HID + 1:_HID + 2])
    bm1p = pair2(bm1.reshape(1, -1))
    W2b = _blkdiag(Wm2).astype(jnp.bfloat16)
    bm2p = pair2(bm2.reshape(1, -1))
    W3p = _blkdiag(Wm3)
    bm3p = pair2(bm3.reshape(1, -1))

    def gru_pack(W):
        return jnp.concatenate([_blkdiag(W[0:_HID]),
                                _blkdiag(W[_HID:2 * _HID]),
                                _blkdiag(W[2 * _HID:])], axis=0)

    Wzp, bzp = gru_pack(Wz), pair2(bz.reshape(1, -1))
    Wrp, brp = gru_pack(Wr), pair2(br.reshape(1, -1))
    Whp, bhp = gru_pack(Wh), pair2(bh.reshape(1, -1))

    Wo1h2 = _blkdiag(Wo1[0:_HID])
    Wo1b2 = _blkdiag(Wo1[_HID:])
    bo1p = pair2(bo1.reshape(1, -1))
    Wo2b = _blkdiag(Wo2)
    bo2p = pair2(bo2.reshape(1, -1))
    Wo3b = _blkdiag(Wo3)
    bo3p = pair2(bo3.reshape(1, -1))

    weights = (w_bin, wJ, Wh2, wbout2, bm1p, W2b, bm2p, W3p, bm3p,
               Wzp, bzp, Wrp, brp, Whp, bhp,
               Wo1h2, Wo1b2, bo1p, Wo2b, bo2p, Wo3b, bo3p)

    def wspec(w):
        return pl.BlockSpec(w.shape, lambda i: (0,) * w.ndim)

    out = pl.pallas_call(
        functools.partial(_gnn_kernel, n_i_tile=64),
        grid=(B,),
        in_specs=[pl.BlockSpec((1, n, nh), lambda i: (i, 0, 0)),
                  pl.BlockSpec((1, n, nh), lambda i: (i, 0, 0)),
                  pl.BlockSpec((1, n, 1), lambda i: (i, 0, 0)),
                  pl.BlockSpec((1, nh, 2), lambda i: (i, 0, 0))]
                 + [wspec(w) for w in weights],
        out_specs=pl.BlockSpec((1, nh, 2), lambda i: (i, 0, 0)),
        out_shape=jax.ShapeDtypeStruct((B, nh, 2), f32),
        scratch_shapes=[pltpu.VMEM((nh, 2 * _HID), f32),
                        pltpu.VMEM((nh, 2 * _HID), f32),
                        pltpu.VMEM((nh, 2 * _HID), f32),
                        pltpu.VMEM((n, nh, 128), jnp.bfloat16)],
        compiler_params=pltpu.CompilerParams(
            dimension_semantics=("parallel",)),
    )(Je, Jo, b, bp, *weights)
    return out.reshape(B, 1, n)


# speedup vs baseline: 1.4332x; 1.0610x over previous
import functools

import jax
import jax.numpy as jnp
from jax.experimental import pallas as pl
from jax.experimental.pallas import tpu as pltpu

_HID = 5
_STEPS = 10


def _gnn_kernel(Je_ref, Jo_ref, b_ref, bp_ref, w_bin_ref, wJ_ref, Wh2_ref,
                wbout2_ref, bm1p_ref, W2b_ref, bm2p_ref, W3p_ref, bm3p_ref,
                Wzp_ref, bzp_ref, Wrp_ref, brp_ref, Whp_ref, bhp_ref,
                Wo1h2_ref, Wo1b2_ref, bo1p_ref, Wo2b_ref, bo2p_ref,
                Wo3b_ref, bo3p_ref,
                out_ref, h0_ref, h1_ref, m0_ref, base_ref, *, n_i_tile):
    f32 = jnp.float32
    bf16 = jnp.bfloat16
    Je = Je_ref[0]
    Jo = Jo_ref[0]
    bv = b_ref[0]
    bp = bp_ref[0]
    n = Je.shape[0]
    nh = n // 2
    TI = n_i_tile

    w_bin = w_bin_ref[:]
    wJ = wJ_ref[:]
    Wh2 = Wh2_ref[:]
    wbout2 = wbout2_ref[:]
    bm1p = bm1p_ref[:]
    W2b = W2b_ref[:]
    bm2p = bm2p_ref[:]
    W3p = W3p_ref[:]
    bm3p = bm3p_ref[:]
    Wzp, bzp = Wzp_ref[:], bzp_ref[:]
    Wrp, brp = Wrp_ref[:], brp_ref[:]
    Whp, bhp = Whp_ref[:], bhp_ref[:]

    v = bv @ w_bin
    vv = jnp.concatenate([v, v], axis=1)
    c2 = bp @ wbout2 + bm1p

    for t in range(n // TI):
        i0 = t * TI
        te = Je[i0:i0 + TI][:, :, None] * wJ[0][None, None, :]
        to = Jo[i0:i0 + TI][:, :, None] * wJ[0][None, None, :]
        base_ref[i0:i0 + TI] = (
            jnp.concatenate([te, to], axis=2)
            + vv[i0:i0 + TI, None, :]).astype(bf16)

    def msg_pair(h0p, h1p):
        u2 = jnp.stack([h0p @ Wh2 + c2, h1p @ Wh2 + c2]).astype(bf16)
        s = jnp.zeros((2, nh, 128), f32)
        for t in range(n // TI):
            i0 = t * TI
            base = base_ref[i0:i0 + TI]
            x1 = jnp.maximum(u2[:, None, :, :] + base[None], 0)
            x2 = jnp.maximum(
                jnp.dot(x1.reshape(2 * TI * nh, 128), W2b,
                        preferred_element_type=f32) + bm2p, 0.0)
            s = s + x2.reshape(2, TI, nh, 128).sum(axis=1)
        d2 = s.reshape(2 * nh, 128) @ W3p + jnp.float32(n) * bm3p
        return d2[0:nh], d2[nh:2 * nh]

    def gru(hp, m0p, m1p):
        ap = jnp.concatenate([hp, m0p, m1p], axis=1)
        z = jax.nn.sigmoid(ap @ Wzp + bzp)
        r = jax.nn.sigmoid(ap @ Wrp + brp)
        jp = jnp.concatenate([r * hp, m0p, m1p], axis=1)
        hh = jnp.tanh(jp @ Whp + bhp)
        return (1.0 - z) * hp + z * hh

    h0_ref[:] = jnp.zeros((nh, 2 * _HID), f32)
    h1_ref[:] = jnp.zeros((nh, 2 * _HID), f32)
    m0_ref[:] = jnp.zeros((nh, 2 * _HID), f32)

    def step(_, tok):
        h0p, h1p = h0_ref[:], h1_ref[:]
        d0, d1 = msg_pair(h0p, h1p)
        m0p = m0_ref[:] + d0
        m1p = m0p + d1
        m0_ref[:] = m0p
        h0_ref[:] = h1p
        h1_ref[:] = gru(h1p, m0p, m1p)
        return tok

    jax.lax.fori_loop(0, _STEPS, step, 0)
    h1p = h1_ref[:]

    Wo1h2 = Wo1h2_ref[:]
    Wo1b2 = Wo1b2_ref[:]
    bo1p = bo1p_ref[:]
    Wo2b = Wo2b_ref[:]
    bo2p = bo2p_ref[:]
    Wo3b = Wo3b_ref[:]
    bo3p = bo3p_ref[:]
    x = jnp.maximum(h1p @ Wo1h2 + bp @ Wo1b2 + bo1p, 0.0)
    x = jnp.maximum(x @ Wo2b + bo2p, 0.0)
    out_ref[0, :, :] = jax.nn.sigmoid(x @ Wo3b + bo3p)


def _blkdiag(a):
    za = jnp.zeros_like(a)
    return jnp.concatenate(
        [jnp.concatenate([a, za], axis=1),
         jnp.concatenate([za, a], axis=1)], axis=0)


def kernel(J, b, Wm1, bm1, Wm2, bm2, Wm3, bm3, Wz, bz, Wr, br, Wh, bh,
           Wo1, bo1, Wo2, bo2, Wo3, bo3):
    B, n = J.shape[0], J.shape[1]
    nh = n // 2
    f32 = jnp.float32

    Je = J[:, :, 0::2]
    Jo = J[:, :, 1::2]
    bp = b.reshape(B, nh, 2)

    def pair2(row):
        return jnp.concatenate([row, row], axis=1)

    w_bin = Wm1[_HID:_HID + 1]
    wJ = Wm1[_HID + 2:_HID + 3] - Wm1[_HID + 3:_HID + 4]
    Wh2 = _blkdiag(Wm1[0:_HID])
    wbout2 = _blkdiag(Wm1[_HID + 1:_HID + 2])
    bm1p = pair2(bm1.reshape(1, -1))
    W2b = _blkdiag(Wm2).astype(jnp.bfloat16)
    bm2p = pair2(bm2.reshape(1, -1))
    W3p = _blkdiag(Wm3)
    bm3p = pair2(bm3.reshape(1, -1))

    def gru_pack(W):
        return jnp.concatenate([_blkdiag(W[0:_HID]),
                                _blkdiag(W[_HID:2 * _HID]),
                                _blkdiag(W[2 * _HID:])], axis=0)

    Wzp, bzp = gru_pack(Wz), pair2(bz.reshape(1, -1))
    Wrp, brp = gru_pack(Wr), pair2(br.reshape(1, -1))
    Whp, bhp = gru_pack(Wh), pair2(bh.reshape(1, -1))

    Wo1h2 = _blkdiag(Wo1[0:_HID])
    Wo1b2 = _blkdiag(Wo1[_HID:])
    bo1p = pair2(bo1.reshape(1, -1))
    Wo2b = _blkdiag(Wo2)
    bo2p = pair2(bo2.reshape(1, -1))
    Wo3b = _blkdiag(Wo3)
    bo3p = pair2(bo3.reshape(1, -1))

    weights = (w_bin, wJ, Wh2, wbout2, bm1p, W2b, bm2p, W3p, bm3p,
               Wzp, bzp, Wrp, brp, Whp, bhp,
               Wo1h2, Wo1b2, bo1p, Wo2b, bo2p, Wo3b, bo3p)

    def wspec(w):
        return pl.BlockSpec(w.shape, lambda i: (0,) * w.ndim)

    out = pl.pallas_call(
        functools.partial(_gnn_kernel, n_i_tile=64),
        grid=(B,),
        in_specs=[pl.BlockSpec((1, n, nh), lambda i: (i, 0, 0)),
                  pl.BlockSpec((1, n, nh), lambda i: (i, 0, 0)),
                  pl.BlockSpec((1, n, 1), lambda i: (i, 0, 0)),
                  pl.BlockSpec((1, nh, 2), lambda i: (i, 0, 0))]
                 + [wspec(w) for w in weights],
        out_specs=pl.BlockSpec((1, nh, 2), lambda i: (i, 0, 0)),
        out_shape=jax.ShapeDtypeStruct((B, nh, 2), f32),
        scratch_shapes=[pltpu.VMEM((nh, 2 * _HID), f32),
                        pltpu.VMEM((nh, 2 * _HID), f32),
                        pltpu.VMEM((nh, 2 * _HID), f32),
                        pltpu.VMEM((n, nh, 128), jnp.bfloat16)],
        compiler_params=pltpu.CompilerParams(
            dimension_semantics=("parallel",)),
    )(Je, Jo, b, bp, *weights)
    return out.reshape(B, 1, n)
